# baseline (device time: 225284 ns/iter reference)
import jax
import jax.numpy as jnp
from jax import lax
from jax.experimental import pallas as pl
from jax.experimental.pallas import tpu as pltpu

M = 4096
D = 4096
HALF = M // 2
CH = 128
NC = HALF // CH
NSTAGE = 4
LOOKAHEAD = 3


def _body(partial_ref, gamma_ref, out_ref,
          mine_buf, th_buf, send_x, recv_x, out_stage,
          mine_sems, th_sems, send_x_sems, recv_x_sems,
          out_copy_sems, send_y_sems, recv_y_sems):
    my_x = lax.axis_index("x")
    my_y = lax.axis_index("y")
    my_z = lax.axis_index("z")
    xp = (1 - my_x, my_y, my_z)
    yp = (my_x, 1 - my_y, my_z)
    my_row0 = my_y * HALF
    fwd_row0 = (1 - my_y) * HALF
    mine_row0 = my_x * M + my_y * HALF
    theirs_row0 = (1 - my_x) * M + my_y * HALF

    barrier = pltpu.get_barrier_semaphore()
    for nbr in (xp, yp):
        pl.semaphore_signal(barrier, inc=1, device_id=nbr,
                            device_id_type=pl.DeviceIdType.MESH)

    def mine_cp(c):
        return pltpu.make_async_copy(
            partial_ref.at[pl.ds(mine_row0 + c * CH, CH), :],
            mine_buf.at[c % 2],
            mine_sems.at[c],
        )

    def th_cp(c):
        return pltpu.make_async_copy(
            partial_ref.at[pl.ds(theirs_row0 + c * CH, CH), :],
            th_buf.at[c % 2],
            th_sems.at[c],
        )

    x_rdmas = {}

    def prep_chunk(j):
        th_cp(j).wait()
        send_x[j] = th_buf[j % 2].astype(jnp.bfloat16)
        if j + 2 < NC:
            th_cp(j + 2).start()

    def launch_chunk(j):
        r = pltpu.make_async_remote_copy(
            src_ref=send_x.at[j],
            dst_ref=recv_x.at[j],
            send_sem=send_x_sems.at[j],
            recv_sem=recv_x_sems.at[j],
            device_id=xp,
            device_id_type=pl.DeviceIdType.MESH,
        )
        r.start()
        x_rdmas[j] = r

    def send_side(j):
        prep_chunk(j)
        launch_chunk(j)

    for c in range(min(2, NC)):
        th_cp(c).start()
        mine_cp(c).start()
    for j in range(min(LOOKAHEAD, NC)):
        prep_chunk(j)
    pl.semaphore_wait(barrier, 2)
    for j in range(min(LOOKAHEAD, NC)):
        launch_chunk(j)

    out_copies = []
    y_rdmas = []
    for c in range(NC):
        if c + LOOKAHEAD < NC:
            send_side(c + LOOKAHEAD)
        s = c % NSTAGE
        if c >= NSTAGE:
            out_copies[c - NSTAGE].wait()
            y_rdmas[c - NSTAGE].wait_send()
        x_rdmas[c].wait_recv()
        mine_cp(c).wait()
        ysum = mine_buf[c % 2] + recv_x[c].astype(jnp.float32)
        ms = jnp.mean(ysum * ysum, axis=-1, keepdims=True)
        res = ysum * lax.rsqrt(ms + 1e-6) * gamma_ref[:, :]
        out_stage[s] = res.astype(jnp.bfloat16)
        if c + 2 < NC:
            mine_cp(c + 2).start()

        cp = pltpu.make_async_copy(
            out_stage.at[s],
            out_ref.at[pl.ds(my_row0 + c * CH, CH), :],
            out_copy_sems.at[c],
        )
        cp.start()
        out_copies.append(cp)

        r = pltpu.make_async_remote_copy(
            src_ref=out_stage.at[s],
            dst_ref=out_ref.at[pl.ds(my_row0 + c * CH, CH), :],
            send_sem=send_y_sems.at[c],
            recv_sem=recv_y_sems.at[c],
            device_id=yp,
            device_id_type=pl.DeviceIdType.MESH,
        )
        r.start()
        y_rdmas.append(r)

    for c in range(NC):
        x_rdmas[c].wait_send()
    for c in range(max(0, NC - NSTAGE), NC):
        out_copies[c].wait()
        y_rdmas[c].wait_send()
    for c in range(NC):
        recv = pltpu.make_async_remote_copy(
            src_ref=out_stage.at[c % NSTAGE],
            dst_ref=out_ref.at[pl.ds(fwd_row0 + c * CH, CH), :],
            send_sem=send_y_sems.at[c],
            recv_sem=recv_y_sems.at[c],
            device_id=yp,
            device_id_type=pl.DeviceIdType.MESH,
        )
        recv.wait_recv()


def kernel(partial, gamma):
    rows = partial.reshape(8192, D)
    gamma2 = gamma.reshape(1, D)

    return pl.pallas_call(
        _body,
        in_specs=[
            pl.BlockSpec(memory_space=pl.ANY),
            pl.BlockSpec(memory_space=pltpu.VMEM),
        ],
        out_specs=pl.BlockSpec(memory_space=pl.ANY),
        out_shape=jax.ShapeDtypeStruct((M, D), jnp.bfloat16),
        scratch_shapes=[
            pltpu.VMEM((2, CH, D), jnp.float32),
            pltpu.VMEM((2, CH, D), jnp.float32),
            pltpu.VMEM((NC, CH, D), jnp.bfloat16),
            pltpu.VMEM((NC, CH, D), jnp.bfloat16),
            pltpu.VMEM((NSTAGE, CH, D), jnp.bfloat16),
            pltpu.SemaphoreType.DMA((NC,)),
            pltpu.SemaphoreType.DMA((NC,)),
            pltpu.SemaphoreType.DMA((NC,)),
            pltpu.SemaphoreType.DMA((NC,)),
            pltpu.SemaphoreType.DMA((NC,)),
            pltpu.SemaphoreType.DMA((NC,)),
            pltpu.SemaphoreType.DMA((NC,)),
        ],
        compiler_params=pltpu.CompilerParams(
            collective_id=0,
            vmem_limit_bytes=56 * 1024 * 1024,
        ),
    )(rows, gamma2)


# device time: 218955 ns/iter; 1.0289x vs baseline; 1.0289x over previous
import jax
import jax.numpy as jnp
from jax import lax
from jax.experimental import pallas as pl
from jax.experimental.pallas import tpu as pltpu

M = 4096
D = 4096
HALF = M // 2
CH = 64
NC = HALF // CH
NSTAGE = 4
LOOKAHEAD = 3


def _body(partial_ref, gamma_ref, out_ref,
          mine_buf, th_buf, send_x, recv_x, out_stage,
          mine_sems, th_sems, send_x_sems, recv_x_sems,
          out_copy_sems, send_y_sems, recv_y_sems):
    my_x = lax.axis_index("x")
    my_y = lax.axis_index("y")
    my_z = lax.axis_index("z")
    xp = (1 - my_x, my_y, my_z)
    yp = (my_x, 1 - my_y, my_z)
    my_row0 = my_y * HALF
    fwd_row0 = (1 - my_y) * HALF
    mine_row0 = my_x * M + my_y * HALF
    theirs_row0 = (1 - my_x) * M + my_y * HALF

    barrier = pltpu.get_barrier_semaphore()
    for nbr in (xp, yp):
        pl.semaphore_signal(barrier, inc=1, device_id=nbr,
                            device_id_type=pl.DeviceIdType.MESH)

    def mine_cp(c):
        return pltpu.make_async_copy(
            partial_ref.at[pl.ds(mine_row0 + c * CH, CH), :],
            mine_buf.at[c % 2],
            mine_sems.at[c],
        )

    def th_cp(c):
        return pltpu.make_async_copy(
            partial_ref.at[pl.ds(theirs_row0 + c * CH, CH), :],
            th_buf.at[c % 2],
            th_sems.at[c],
        )

    x_rdmas = {}

    def prep_chunk(j):
        th_cp(j).wait()
        send_x[j] = th_buf[j % 2].astype(jnp.bfloat16)
        if j + 2 < NC:
            th_cp(j + 2).start()

    def launch_chunk(j):
        r = pltpu.make_async_remote_copy(
            src_ref=send_x.at[j],
            dst_ref=recv_x.at[j],
            send_sem=send_x_sems.at[j],
            recv_sem=recv_x_sems.at[j],
            device_id=xp,
            device_id_type=pl.DeviceIdType.MESH,
        )
        r.start()
        x_rdmas[j] = r

    def send_side(j):
        prep_chunk(j)
        launch_chunk(j)

    for c in range(min(2, NC)):
        th_cp(c).start()
        mine_cp(c).start()
    for j in range(min(LOOKAHEAD, NC)):
        prep_chunk(j)
    pl.semaphore_wait(barrier, 2)
    for j in range(min(LOOKAHEAD, NC)):
        launch_chunk(j)

    out_copies = []
    y_rdmas = []
    for c in range(NC):
        if c + LOOKAHEAD < NC:
            send_side(c + LOOKAHEAD)
        s = c % NSTAGE
        if c >= NSTAGE:
            out_copies[c - NSTAGE].wait()
            y_rdmas[c - NSTAGE].wait_send()
        x_rdmas[c].wait_recv()
        mine_cp(c).wait()
        ysum = mine_buf[c % 2] + recv_x[c].astype(jnp.float32)
        ms = jnp.mean(ysum * ysum, axis=-1, keepdims=True)
        res = ysum * lax.rsqrt(ms + 1e-6) * gamma_ref[:, :]
        out_stage[s] = res.astype(jnp.bfloat16)
        if c + 2 < NC:
            mine_cp(c + 2).start()

        cp = pltpu.make_async_copy(
            out_stage.at[s],
            out_ref.at[pl.ds(my_row0 + c * CH, CH), :],
            out_copy_sems.at[c],
        )
        cp.start()
        out_copies.append(cp)

        r = pltpu.make_async_remote_copy(
            src_ref=out_stage.at[s],
            dst_ref=out_ref.at[pl.ds(my_row0 + c * CH, CH), :],
            send_sem=send_y_sems.at[c],
            recv_sem=recv_y_sems.at[c],
            device_id=yp,
            device_id_type=pl.DeviceIdType.MESH,
        )
        r.start()
        y_rdmas.append(r)

    for c in range(NC):
        x_rdmas[c].wait_send()
    for c in range(max(0, NC - NSTAGE), NC):
        out_copies[c].wait()
        y_rdmas[c].wait_send()
    for c in range(NC):
        recv = pltpu.make_async_remote_copy(
            src_ref=out_stage.at[c % NSTAGE],
            dst_ref=out_ref.at[pl.ds(fwd_row0 + c * CH, CH), :],
            send_sem=send_y_sems.at[c],
            recv_sem=recv_y_sems.at[c],
            device_id=yp,
            device_id_type=pl.DeviceIdType.MESH,
        )
        recv.wait_recv()


def kernel(partial, gamma):
    rows = partial.reshape(8192, D)
    gamma2 = gamma.reshape(1, D)

    return pl.pallas_call(
        _body,
        in_specs=[
            pl.BlockSpec(memory_space=pl.ANY),
            pl.BlockSpec(memory_space=pltpu.VMEM),
        ],
        out_specs=pl.BlockSpec(memory_space=pl.ANY),
        out_shape=jax.ShapeDtypeStruct((M, D), jnp.bfloat16),
        scratch_shapes=[
            pltpu.VMEM((2, CH, D), jnp.float32),
            pltpu.VMEM((2, CH, D), jnp.float32),
            pltpu.VMEM((NC, CH, D), jnp.bfloat16),
            pltpu.VMEM((NC, CH, D), jnp.bfloat16),
            pltpu.VMEM((NSTAGE, CH, D), jnp.bfloat16),
            pltpu.SemaphoreType.DMA((NC,)),
            pltpu.SemaphoreType.DMA((NC,)),
            pltpu.SemaphoreType.DMA((NC,)),
            pltpu.SemaphoreType.DMA((NC,)),
            pltpu.SemaphoreType.DMA((NC,)),
            pltpu.SemaphoreType.DMA((NC,)),
            pltpu.SemaphoreType.DMA((NC,)),
        ],
        compiler_params=pltpu.CompilerParams(
            collective_id=0,
            vmem_limit_bytes=56 * 1024 * 1024,
        ),
    )(rows, gamma2)


# device time: 180238 ns/iter; 1.2499x vs baseline; 1.2148x over previous
import jax
import jax.numpy as jnp
from jax import lax
from jax.experimental import pallas as pl
from jax.experimental.pallas import tpu as pltpu

M = 4096
D = 4096
Q = M // 4
CH = 128
NCQ = Q // CH
HR = NCQ // 2
NSTAGE = 4
LOOKAHEAD = 3


def _body(partial_ref, gamma_ref, out_ref,
          mine_buf, th_buf, send_x, recv_x, out_stage,
          mine_sems, th_sems, send_x_sems, recv_x_sems, out_copy_sems,
          y_send_own, y_send_rel, y_recv_own, y_recv_rel,
          z_send_own, z_send_rel, z_recv_own, z_recv_rel):
    my_x = lax.axis_index("x")
    my_y = lax.axis_index("y")
    my_z = lax.axis_index("z")
    zr = lax.rem(my_z, 2)
    pz = my_z + 1 - 2 * zr
    xp = (1 - my_x, my_y, my_z)
    yp = (my_x, 1 - my_y, my_z)
    zp = (my_x, my_y, pz)

    qD = 2 * my_y + zr
    qY = 2 * (1 - my_y) + zr
    qZ = 2 * my_y + (1 - zr)
    qX2 = 2 * (1 - my_y) + (1 - zr)

    my_row0 = qD * Q
    mine_row0 = my_x * M + qD * Q
    theirs_row0 = (1 - my_x) * M + qD * Q

    barrier = pltpu.get_barrier_semaphore()
    for nbr in (xp, yp, zp):
        pl.semaphore_signal(barrier, inc=1, device_id=nbr,
                            device_id_type=pl.DeviceIdType.MESH)

    def mine_cp(c):
        return pltpu.make_async_copy(
            partial_ref.at[pl.ds(mine_row0 + c * CH, CH), :],
            mine_buf.at[c % 2],
            mine_sems.at[c],
        )

    def th_cp(c):
        return pltpu.make_async_copy(
            partial_ref.at[pl.ds(theirs_row0 + c * CH, CH), :],
            th_buf.at[c % 2],
            th_sems.at[c],
        )

    x_rdmas = {}

    def prep_chunk(j):
        th_cp(j).wait()
        send_x[j] = th_buf[j % 2].astype(jnp.bfloat16)
        if j + 2 < NCQ:
            th_cp(j + 2).start()

    def launch_chunk(j):
        r = pltpu.make_async_remote_copy(
            src_ref=send_x.at[j],
            dst_ref=recv_x.at[j],
            send_sem=send_x_sems.at[j],
            recv_sem=recv_x_sems.at[j],
            device_id=xp,
            device_id_type=pl.DeviceIdType.MESH,
        )
        r.start()
        x_rdmas[j] = r

    for c in range(min(2, NCQ)):
        th_cp(c).start()
        mine_cp(c).start()
    for j in range(min(LOOKAHEAD, NCQ)):
        prep_chunk(j)
    pl.semaphore_wait(barrier, 3)
    for j in range(min(LOOKAHEAD, NCQ)):
        launch_chunk(j)

    def own_rdma(c, s, peer, send_sems):
        return pltpu.make_async_remote_copy(
            src_ref=out_stage.at[s],
            dst_ref=out_ref.at[pl.ds(my_row0 + c * CH, CH), :],
            send_sem=send_sems.at[c],
            recv_sem=y_recv_own.at[c] if peer is yp else z_recv_own.at[c],
            device_id=peer,
            device_id_type=pl.DeviceIdType.MESH,
        )

    out_copies = []
    yz_rdmas = []
    for c in range(NCQ):
        if c + LOOKAHEAD < NCQ:
            prep_chunk(c + LOOKAHEAD)
            launch_chunk(c + LOOKAHEAD)
        s = c % NSTAGE
        if c >= NSTAGE:
            out_copies[c - NSTAGE].wait()
            yz_rdmas[2 * (c - NSTAGE)].wait_send()
            yz_rdmas[2 * (c - NSTAGE) + 1].wait_send()
        x_rdmas[c].wait_recv()
        mine_cp(c).wait()
        ysum = mine_buf[c % 2] + recv_x[c].astype(jnp.float32)
        ms = jnp.mean(ysum * ysum, axis=-1, keepdims=True)
        res = ysum * lax.rsqrt(ms + 1e-6) * gamma_ref[:, :]
        out_stage[s] = res.astype(jnp.bfloat16)
        if c + 2 < NCQ:
            mine_cp(c + 2).start()

        cp = pltpu.make_async_copy(
            out_stage.at[s],
            out_ref.at[pl.ds(my_row0 + c * CH, CH), :],
            out_copy_sems.at[c],
        )
        cp.start()
        out_copies.append(cp)
        ry = own_rdma(c, s, yp, y_send_own)
        rz = own_rdma(c, s, zp, z_send_own)
        ry.start()
        rz.start()
        yz_rdmas += [ry, rz]

    for c in range(NCQ):
        x_rdmas[c].wait_send()

    def recv_own_desc(c, peer, row0, send_sems, recv_sems):
        return pltpu.make_async_remote_copy(
            src_ref=out_ref.at[pl.ds(row0 + c * CH, CH), :],
            dst_ref=out_ref.at[pl.ds(row0 + c * CH, CH), :],
            send_sem=send_sems.at[c],
            recv_sem=recv_sems.at[c],
            device_id=peer,
            device_id_type=pl.DeviceIdType.MESH,
        )

    rel_rdmas = []
    for r in range(HR):
        recv_own_desc(r, zp, qZ * Q, z_send_own, z_recv_own).wait_recv()
        rel = pltpu.make_async_remote_copy(
            src_ref=out_ref.at[pl.ds(qZ * Q + r * CH, CH), :],
            dst_ref=out_ref.at[pl.ds(qZ * Q + r * CH, CH), :],
            send_sem=y_send_rel.at[r],
            recv_sem=y_recv_rel.at[r],
            device_id=yp,
            device_id_type=pl.DeviceIdType.MESH,
        )
        rel.start()
        rel_rdmas.append(rel)
    for r in range(HR, NCQ):
        recv_own_desc(r, yp, qY * Q, y_send_own, y_recv_own).wait_recv()
        rel = pltpu.make_async_remote_copy(
            src_ref=out_ref.at[pl.ds(qY * Q + r * CH, CH), :],
            dst_ref=out_ref.at[pl.ds(qY * Q + r * CH, CH), :],
            send_sem=z_send_rel.at[r - HR],
            recv_sem=z_recv_rel.at[r - HR],
            device_id=zp,
            device_id_type=pl.DeviceIdType.MESH,
        )
        rel.start()
        rel_rdmas.append(rel)

    for c in range(HR):
        recv_own_desc(c, yp, qY * Q, y_send_own, y_recv_own).wait_recv()
    for c in range(HR, NCQ):
        recv_own_desc(c, zp, qZ * Q, z_send_own, z_recv_own).wait_recv()
    for r in range(HR):
        recv_own_desc(r, yp, qX2 * Q, y_send_rel, y_recv_rel).wait_recv()
        recv_own_desc(r, zp, qX2 * Q + HR * CH,
                      z_send_rel, z_recv_rel).wait_recv()
    for rel in rel_rdmas:
        rel.wait_send()
    for c in range(max(0, NCQ - NSTAGE), NCQ):
        out_copies[c].wait()
        yz_rdmas[2 * c].wait_send()
        yz_rdmas[2 * c + 1].wait_send()


def kernel(partial, gamma):
    rows = partial.reshape(8192, D)
    gamma2 = gamma.reshape(1, D)

    return pl.pallas_call(
        _body,
        in_specs=[
            pl.BlockSpec(memory_space=pl.ANY),
            pl.BlockSpec(memory_space=pltpu.VMEM),
        ],
        out_specs=pl.BlockSpec(memory_space=pltpu.MemorySpace.HBM),
        out_shape=jax.ShapeDtypeStruct((M, D), jnp.bfloat16),
        scratch_shapes=[
            pltpu.VMEM((2, CH, D), jnp.float32),
            pltpu.VMEM((2, CH, D), jnp.float32),
            pltpu.VMEM((NCQ, CH, D), jnp.bfloat16),
            pltpu.VMEM((NCQ, CH, D), jnp.bfloat16),
            pltpu.VMEM((NSTAGE, CH, D), jnp.bfloat16),
            pltpu.SemaphoreType.DMA((NCQ,)),
            pltpu.SemaphoreType.DMA((NCQ,)),
            pltpu.SemaphoreType.DMA((NCQ,)),
            pltpu.SemaphoreType.DMA((NCQ,)),
            pltpu.SemaphoreType.DMA((NCQ,)),
            pltpu.SemaphoreType.DMA((NCQ,)),
            pltpu.SemaphoreType.DMA((HR,)),
            pltpu.SemaphoreType.DMA((NCQ,)),
            pltpu.SemaphoreType.DMA((HR,)),
            pltpu.SemaphoreType.DMA((NCQ,)),
            pltpu.SemaphoreType.DMA((HR,)),
            pltpu.SemaphoreType.DMA((NCQ,)),
            pltpu.SemaphoreType.DMA((HR,)),
        ],
        compiler_params=pltpu.CompilerParams(
            collective_id=0,
            vmem_limit_bytes=56 * 1024 * 1024,
        ),
    )(rows, gamma2)
